# baseline (speedup 1.0000x reference)
import functools

import jax
import jax.numpy as jnp
from jax import lax
from jax.experimental import pallas as pl
from jax.experimental.pallas import tpu as pltpu
from jax.experimental.pallas import tpu_sc as plsc

B = 16384
D = 64
H1 = 128
EPS = 1e-5

NC = 2
NS = 16
NW = NC * NS
BPW = B // NW
CHUNK = 128
NCHUNK = BPW // CHUNK

RB = 2048


def _make_gather():
    mesh = plsc.VectorSubcoreMesh(core_axis_name="c", subcore_axis_name="s")

    @functools.partial(
        pl.kernel,
        mesh=mesh,
        compiler_params=pltpu.CompilerParams(use_tc_tiling_on_sc=False),
        out_type=[
            jax.ShapeDtypeStruct((B, D), jnp.float32),
            jax.ShapeDtypeStruct((B, D), jnp.float32),
        ],
        scratch_types=[
            pltpu.VMEM((NCHUNK, CHUNK), jnp.int32),
            pltpu.VMEM((NCHUNK, CHUNK), jnp.int32),
            pltpu.VMEM((BPW, D), jnp.float32),
            pltpu.VMEM((BPW, D), jnp.float32),
            pltpu.SemaphoreType.DMA,
            pltpu.SemaphoreType.DMA,
        ],
    )
    def gather(uidx_hbm, iidx_hbm, utab_hbm, itab_hbm, ue_hbm, ie_hbm,
               uidx_v, iidx_v, urows_v, irows_v, usem, isem):
        wid = lax.axis_index("s") * NC + lax.axis_index("c")
        base = wid * BPW
        pltpu.sync_copy(uidx_hbm.at[wid], uidx_v)
        pltpu.sync_copy(iidx_hbm.at[wid], iidx_v)
        copies = []
        for j in range(NCHUNK):
            copies.append(pltpu.async_copy(
                utab_hbm.at[uidx_v.at[j]],
                urows_v.at[pl.ds(j * CHUNK, CHUNK)], usem))
            copies.append(pltpu.async_copy(
                itab_hbm.at[iidx_v.at[j]],
                irows_v.at[pl.ds(j * CHUNK, CHUNK)], isem))
        for c in copies:
            c.wait()
        pltpu.sync_copy(urows_v, ue_hbm.at[pl.ds(base, BPW)])
        pltpu.sync_copy(irows_v, ie_hbm.at[pl.ds(base, BPW)])

    return gather


_gather = _make_gather()


def _mlp_body(ue, ie, a1u, a1i, c1, a2, c2, w3, b3, out):
    h = jnp.dot(ue[...], a1u[...], preferred_element_type=jnp.float32)
    h = h + jnp.dot(ie[...], a1i[...], preferred_element_type=jnp.float32)
    h = jnp.maximum(h + c1[...], 0.0)
    h = jnp.dot(h, a2[...], preferred_element_type=jnp.float32)
    h = jnp.maximum(h + c2[...], 0.0)
    out[...] = jnp.sum(h * w3[...], axis=1, keepdims=True) + b3[...]


def kernel(user, item, user_table, item_table,
           W1, b1, g1, be1, rm1, rv1,
           W2, b2, g2, be2, rm2, rv2,
           W3, b3):
    uidx = user.astype(jnp.int32).reshape(NW, NCHUNK, CHUNK)
    iidx = item.astype(jnp.int32).reshape(NW, NCHUNK, CHUNK)
    ue, ie = _gather(uidx, iidx, user_table, item_table)

    s1 = g1 * lax.rsqrt(rv1 + EPS)
    a1 = (W1 * s1[:, None]).T
    c1 = ((b1 - rm1) * s1 + be1).reshape(1, H1)
    s2 = g2 * lax.rsqrt(rv2 + EPS)
    a2 = (W2 * s2[:, None]).T
    c2 = ((b2 - rm2) * s2 + be2).reshape(1, D)

    out = pl.pallas_call(
        _mlp_body,
        grid=(B // RB,),
        in_specs=[
            pl.BlockSpec((RB, D), lambda i: (i, 0)),
            pl.BlockSpec((RB, D), lambda i: (i, 0)),
            pl.BlockSpec((D, H1), lambda i: (0, 0)),
            pl.BlockSpec((D, H1), lambda i: (0, 0)),
            pl.BlockSpec((1, H1), lambda i: (0, 0)),
            pl.BlockSpec((H1, D), lambda i: (0, 0)),
            pl.BlockSpec((1, D), lambda i: (0, 0)),
            pl.BlockSpec((1, D), lambda i: (0, 0)),
            pl.BlockSpec((1, 1), lambda i: (0, 0)),
        ],
        out_specs=pl.BlockSpec((RB, 1), lambda i: (i, 0)),
        out_shape=jax.ShapeDtypeStruct((B, 1), jnp.float32),
    )(ue, ie, a1[:D], a1[D:], c1, a2, c2, W3, b3.reshape(1, 1))
    return out.reshape(B)


# speedup vs baseline: 1.6472x; 1.6472x over previous
import functools

import jax
import jax.numpy as jnp
from jax import lax
from jax.experimental import pallas as pl
from jax.experimental.pallas import tpu as pltpu
from jax.experimental.pallas import tpu_sc as plsc

B = 16384
D = 64
H1 = 128
EPS = 1e-5

NC = 2
NS = 16
NW = NC * NS
BPW = B // NW
CHUNK = 128
NCHUNK = BPW // CHUNK

CB = 2048
RB = 2048


def _pack_body(lo, hi, out):
    out[...] = jnp.concatenate([lo[...].T, hi[...].T], axis=1)


def _pack_table(table):
    n = table.shape[0]
    grid = (n + 2 * CB - 1) // (2 * CB)
    last = (n + CB - 1) // CB - 1
    tt = table.T
    return pl.pallas_call(
        _pack_body,
        grid=(grid,),
        in_specs=[
            pl.BlockSpec((D, CB), lambda g: (0, jnp.minimum(2 * g, last))),
            pl.BlockSpec((D, CB), lambda g: (0, jnp.minimum(2 * g + 1, last))),
        ],
        out_specs=pl.BlockSpec((CB, 2 * D), lambda g: (g, 0)),
        out_shape=jax.ShapeDtypeStruct((grid * CB, 2 * D), jnp.float32),
    )(tt, tt)


def _make_gather():
    mesh = plsc.VectorSubcoreMesh(core_axis_name="c", subcore_axis_name="s")

    @functools.partial(
        pl.kernel,
        mesh=mesh,
        compiler_params=pltpu.CompilerParams(use_tc_tiling_on_sc=False),
        out_type=[
            jax.ShapeDtypeStruct((B, 2 * D), jnp.float32),
            jax.ShapeDtypeStruct((B, 2 * D), jnp.float32),
        ],
        scratch_types=[
            pltpu.VMEM((NCHUNK, CHUNK), jnp.int32),
            pltpu.VMEM((NCHUNK, CHUNK), jnp.int32),
            pltpu.VMEM((BPW, 2 * D), jnp.float32),
            pltpu.SemaphoreType.DMA,
        ],
    )
    def gather(uidx_hbm, iidx_hbm, utab_hbm, itab_hbm, ue_hbm, ie_hbm,
               uidx_v, iidx_v, rows_v, sem):
        wid = lax.axis_index("s") * NC + lax.axis_index("c")
        base = wid * BPW
        pltpu.sync_copy(uidx_hbm.at[wid], uidx_v)
        pltpu.sync_copy(iidx_hbm.at[wid], iidx_v)
        ucopies = [
            pltpu.async_copy(utab_hbm.at[uidx_v.at[j]],
                             rows_v.at[pl.ds(j * CHUNK, CHUNK)], sem)
            for j in range(NCHUNK)
        ]
        for c in ucopies:
            c.wait()
        pltpu.sync_copy(rows_v, ue_hbm.at[pl.ds(base, BPW)])
        icopies = [
            pltpu.async_copy(itab_hbm.at[iidx_v.at[j]],
                             rows_v.at[pl.ds(j * CHUNK, CHUNK)], sem)
            for j in range(NCHUNK)
        ]
        for c in icopies:
            c.wait()
        pltpu.sync_copy(rows_v, ie_hbm.at[pl.ds(base, BPW)])

    return gather


_gather_cache = []


def _gather(*args):
    if not _gather_cache:
        _gather_cache.append(_make_gather())
    return _gather_cache[0](*args)


def _mlp_body(ue, ie, mu, mi, a1u, a1i, c1, a2, c2, w3, b3, out):
    h = jnp.dot(ue[...] * mu[...], a1u[...],
                preferred_element_type=jnp.float32)
    h = h + jnp.dot(ie[...] * mi[...], a1i[...],
                    preferred_element_type=jnp.float32)
    h = jnp.maximum(h + c1[...], 0.0)
    h = jnp.dot(h, a2[...], preferred_element_type=jnp.float32)
    h = jnp.maximum(h + c2[...], 0.0)
    out[...] = jnp.sum(h * w3[...], axis=1, keepdims=True) + b3[...]


def kernel(user, item, user_table, item_table,
           W1, b1, g1, be1, rm1, rv1,
           W2, b2, g2, be2, rm2, rv2,
           W3, b3):
    user = user.astype(jnp.int32)
    item = item.astype(jnp.int32)

    ut_p = _pack_table(user_table)
    it_p = _pack_table(item_table)

    upacked = ((user >> 12) << 11) | (user & (CB - 1))
    ipacked = ((item >> 12) << 11) | (item & (CB - 1))
    uidx = upacked.reshape(NW, NCHUNK, CHUNK)
    iidx = ipacked.reshape(NW, NCHUNK, CHUNK)
    ue2, ie2 = _gather(uidx, iidx, ut_p, it_p)

    half = (jnp.arange(2 * D, dtype=jnp.int32) >= D)[None, :]
    mu = (half == ((user[:, None] & CB) != 0)).astype(jnp.float32)
    mi = (half == ((item[:, None] & CB) != 0)).astype(jnp.float32)

    s1 = g1 * lax.rsqrt(rv1 + EPS)
    a1 = (W1 * s1[:, None]).T
    c1 = ((b1 - rm1) * s1 + be1).reshape(1, H1)
    a1u = jnp.concatenate([a1[:D], a1[:D]], axis=0)
    a1i = jnp.concatenate([a1[D:], a1[D:]], axis=0)
    s2 = g2 * lax.rsqrt(rv2 + EPS)
    a2 = (W2 * s2[:, None]).T
    c2 = ((b2 - rm2) * s2 + be2).reshape(1, D)

    out = pl.pallas_call(
        _mlp_body,
        grid=(B // RB,),
        in_specs=[
            pl.BlockSpec((RB, 2 * D), lambda i: (i, 0)),
            pl.BlockSpec((RB, 2 * D), lambda i: (i, 0)),
            pl.BlockSpec((RB, 2 * D), lambda i: (i, 0)),
            pl.BlockSpec((RB, 2 * D), lambda i: (i, 0)),
            pl.BlockSpec((2 * D, H1), lambda i: (0, 0)),
            pl.BlockSpec((2 * D, H1), lambda i: (0, 0)),
            pl.BlockSpec((1, H1), lambda i: (0, 0)),
            pl.BlockSpec((H1, D), lambda i: (0, 0)),
            pl.BlockSpec((1, D), lambda i: (0, 0)),
            pl.BlockSpec((1, D), lambda i: (0, 0)),
            pl.BlockSpec((1, 1), lambda i: (0, 0)),
        ],
        out_specs=pl.BlockSpec((RB, 1), lambda i: (i, 0)),
        out_shape=jax.ShapeDtypeStruct((B, 1), jnp.float32),
    )(ue2, ie2, mu, mi, a1u, a1i, c1, a2, c2, W3, b3.reshape(1, 1))
    return out.reshape(B)


# speedup vs baseline: 1.9350x; 1.1747x over previous
import functools

import jax
import jax.numpy as jnp
from jax import lax
from jax.experimental import pallas as pl
from jax.experimental.pallas import tpu as pltpu
from jax.experimental.pallas import tpu_sc as plsc

B = 16384
D = 64
H1 = 128
EPS = 1e-5

NC = 2
NS = 16
NW = NC * NS
BPW = B // NW
CHUNK = 128
NCHUNK = BPW // CHUNK

CB = 2048
RB = 2048


def _pack_body(lo, hi, out):
    out[...] = jnp.concatenate([lo[...], hi[...]], axis=0).T


def _pack_table(table):
    n = table.shape[0]
    grid = (n + 2 * CB - 1) // (2 * CB)
    last = (n + CB - 1) // CB - 1
    tt = table.T
    return pl.pallas_call(
        _pack_body,
        grid=(grid,),
        in_specs=[
            pl.BlockSpec((D, CB), lambda g: (0, jnp.minimum(2 * g, last))),
            pl.BlockSpec((D, CB), lambda g: (0, jnp.minimum(2 * g + 1, last))),
        ],
        out_specs=pl.BlockSpec((CB, 2 * D), lambda g: (g, 0)),
        out_shape=jax.ShapeDtypeStruct((grid * CB, 2 * D), jnp.float32),
    )(tt, tt)


def _make_gather():
    mesh = plsc.VectorSubcoreMesh(core_axis_name="c", subcore_axis_name="s")

    @functools.partial(
        pl.kernel,
        mesh=mesh,
        compiler_params=pltpu.CompilerParams(use_tc_tiling_on_sc=False),
        out_type=[
            jax.ShapeDtypeStruct((B, 2 * D), jnp.float32),
            jax.ShapeDtypeStruct((B, 2 * D), jnp.float32),
        ],
        scratch_types=[
            pltpu.VMEM((NCHUNK, CHUNK), jnp.int32),
            pltpu.VMEM((NCHUNK, CHUNK), jnp.int32),
            pltpu.VMEM((BPW, 2 * D), jnp.float32),
            pltpu.SemaphoreType.DMA,
        ],
    )
    def gather(uidx_hbm, iidx_hbm, utab_hbm, itab_hbm, ue_hbm, ie_hbm,
               uidx_v, iidx_v, rows_v, sem):
        wid = lax.axis_index("s") * NC + lax.axis_index("c")
        base = wid * BPW
        pltpu.sync_copy(uidx_hbm.at[wid], uidx_v)
        pltpu.sync_copy(iidx_hbm.at[wid], iidx_v)
        ucopies = [
            pltpu.async_copy(utab_hbm.at[uidx_v.at[j]],
                             rows_v.at[pl.ds(j * CHUNK, CHUNK)], sem)
            for j in range(NCHUNK)
        ]
        for c in ucopies:
            c.wait()
        pltpu.sync_copy(rows_v, ue_hbm.at[pl.ds(base, BPW)])
        icopies = [
            pltpu.async_copy(itab_hbm.at[iidx_v.at[j]],
                             rows_v.at[pl.ds(j * CHUNK, CHUNK)], sem)
            for j in range(NCHUNK)
        ]
        for c in icopies:
            c.wait()
        pltpu.sync_copy(rows_v, ie_hbm.at[pl.ds(base, BPW)])

    return gather


_gather_cache = []


def _gather(*args):
    if not _gather_cache:
        _gather_cache.append(_make_gather())
    return _gather_cache[0](*args)


def _mlp_body(ue, ie, mu, mi, a1u, a1i, c1, a2, c2, w3, b3, out):
    h = jnp.dot(ue[...] * mu[...], a1u[...],
                preferred_element_type=jnp.float32)
    h = h + jnp.dot(ie[...] * mi[...], a1i[...],
                    preferred_element_type=jnp.float32)
    h = jnp.maximum(h + c1[...], 0.0)
    h = jnp.dot(h, a2[...], preferred_element_type=jnp.float32)
    h = jnp.maximum(h + c2[...], 0.0)
    out[...] = jnp.sum(h * w3[...], axis=1, keepdims=True) + b3[...]


def kernel(user, item, user_table, item_table,
           W1, b1, g1, be1, rm1, rv1,
           W2, b2, g2, be2, rm2, rv2,
           W3, b3):
    user = user.astype(jnp.int32)
    item = item.astype(jnp.int32)

    ut_p = _pack_table(user_table)
    it_p = _pack_table(item_table)

    upacked = ((user >> 12) << 11) | (user & (CB - 1))
    ipacked = ((item >> 12) << 11) | (item & (CB - 1))
    uidx = upacked.reshape(NW, NCHUNK, CHUNK)
    iidx = ipacked.reshape(NW, NCHUNK, CHUNK)
    ue2, ie2 = _gather(uidx, iidx, ut_p, it_p)

    half = (jnp.arange(2 * D, dtype=jnp.int32) >= D)[None, :]
    mu = (half == ((user[:, None] & CB) != 0)).astype(jnp.float32)
    mi = (half == ((item[:, None] & CB) != 0)).astype(jnp.float32)

    s1 = g1 * lax.rsqrt(rv1 + EPS)
    a1 = (W1 * s1[:, None]).T
    c1 = ((b1 - rm1) * s1 + be1).reshape(1, H1)
    a1u = jnp.concatenate([a1[:D], a1[:D]], axis=0)
    a1i = jnp.concatenate([a1[D:], a1[D:]], axis=0)
    s2 = g2 * lax.rsqrt(rv2 + EPS)
    a2 = (W2 * s2[:, None]).T
    c2 = ((b2 - rm2) * s2 + be2).reshape(1, D)

    out = pl.pallas_call(
        _mlp_body,
        grid=(B // RB,),
        in_specs=[
            pl.BlockSpec((RB, 2 * D), lambda i: (i, 0)),
            pl.BlockSpec((RB, 2 * D), lambda i: (i, 0)),
            pl.BlockSpec((RB, 2 * D), lambda i: (i, 0)),
            pl.BlockSpec((RB, 2 * D), lambda i: (i, 0)),
            pl.BlockSpec((2 * D, H1), lambda i: (0, 0)),
            pl.BlockSpec((2 * D, H1), lambda i: (0, 0)),
            pl.BlockSpec((1, H1), lambda i: (0, 0)),
            pl.BlockSpec((H1, D), lambda i: (0, 0)),
            pl.BlockSpec((1, D), lambda i: (0, 0)),
            pl.BlockSpec((1, D), lambda i: (0, 0)),
            pl.BlockSpec((1, 1), lambda i: (0, 0)),
        ],
        out_specs=pl.BlockSpec((RB, 1), lambda i: (i, 0)),
        out_shape=jax.ShapeDtypeStruct((B, 1), jnp.float32),
    )(ue2, ie2, mu, mi, a1u, a1i, c1, a2, c2, W3, b3.reshape(1, 1))
    return out.reshape(B)


# speedup vs baseline: 2.8756x; 1.4861x over previous
import functools

import jax
import jax.numpy as jnp
from jax import lax
from jax.experimental import pallas as pl
from jax.experimental.pallas import tpu as pltpu
from jax.experimental.pallas import tpu_sc as plsc

B = 16384
D = 64
H1 = 128
EPS = 1e-5

NC = 2
NS = 16
NW = NC * NS
BPW = B // NW
CHUNK = 128
NCHUNK = BPW // CHUNK

CB = 16384
SH = CB.bit_length() - 1
RB = 2048


def _pack_body(lo, hi, out):
    out[...] = jnp.concatenate([lo[...], hi[...]], axis=0).T


def _pack_table(table):
    n = table.shape[0]
    grid = (n + 2 * CB - 1) // (2 * CB)
    last = (n + CB - 1) // CB - 1
    tt = table.T
    return pl.pallas_call(
        _pack_body,
        grid=(grid,),
        in_specs=[
            pl.BlockSpec((D, CB), lambda g: (0, jnp.minimum(2 * g, last))),
            pl.BlockSpec((D, CB), lambda g: (0, jnp.minimum(2 * g + 1, last))),
        ],
        out_specs=pl.BlockSpec((CB, 2 * D), lambda g: (g, 0)),
        out_shape=jax.ShapeDtypeStruct((grid * CB, 2 * D), jnp.float32),
    )(tt, tt)


def _make_gather():
    mesh = plsc.VectorSubcoreMesh(core_axis_name="c", subcore_axis_name="s")

    @functools.partial(
        pl.kernel,
        mesh=mesh,
        compiler_params=pltpu.CompilerParams(use_tc_tiling_on_sc=False),
        out_type=[
            jax.ShapeDtypeStruct((B, 2 * D), jnp.float32),
            jax.ShapeDtypeStruct((B, 2 * D), jnp.float32),
        ],
        scratch_types=[
            pltpu.VMEM((NCHUNK, CHUNK), jnp.int32),
            pltpu.VMEM((NCHUNK, CHUNK), jnp.int32),
            pltpu.VMEM((BPW, 2 * D), jnp.float32),
            pltpu.SemaphoreType.DMA,
        ],
    )
    def gather(uidx_hbm, iidx_hbm, utab_hbm, itab_hbm, ue_hbm, ie_hbm,
               uidx_v, iidx_v, rows_v, sem):
        wid = lax.axis_index("s") * NC + lax.axis_index("c")
        base = wid * BPW
        pltpu.sync_copy(uidx_hbm.at[wid], uidx_v)
        pltpu.sync_copy(iidx_hbm.at[wid], iidx_v)
        ucopies = [
            pltpu.async_copy(utab_hbm.at[uidx_v.at[j]],
                             rows_v.at[pl.ds(j * CHUNK, CHUNK)], sem)
            for j in range(NCHUNK)
        ]
        for c in ucopies:
            c.wait()
        pltpu.sync_copy(rows_v, ue_hbm.at[pl.ds(base, BPW)])
        icopies = [
            pltpu.async_copy(itab_hbm.at[iidx_v.at[j]],
                             rows_v.at[pl.ds(j * CHUNK, CHUNK)], sem)
            for j in range(NCHUNK)
        ]
        for c in icopies:
            c.wait()
        pltpu.sync_copy(rows_v, ie_hbm.at[pl.ds(base, BPW)])

    return gather


_gather_cache = []


def _gather(*args):
    if not _gather_cache:
        _gather_cache.append(_make_gather())
    return _gather_cache[0](*args)


def _mlp_body(ue, ie, mu, mi, a1u, a1i, c1, a2, c2, w3, b3, out):
    h = jnp.dot(ue[...] * mu[...], a1u[...],
                preferred_element_type=jnp.float32)
    h = h + jnp.dot(ie[...] * mi[...], a1i[...],
                    preferred_element_type=jnp.float32)
    h = jnp.maximum(h + c1[...], 0.0)
    h = jnp.dot(h, a2[...], preferred_element_type=jnp.float32)
    h = jnp.maximum(h + c2[...], 0.0)
    out[...] = jnp.sum(h * w3[...], axis=1, keepdims=True) + b3[...]


def kernel(user, item, user_table, item_table,
           W1, b1, g1, be1, rm1, rv1,
           W2, b2, g2, be2, rm2, rv2,
           W3, b3):
    user = user.astype(jnp.int32)
    item = item.astype(jnp.int32)

    ut_p = _pack_table(user_table)
    it_p = _pack_table(item_table)

    upacked = ((user >> (SH + 1)) << SH) | (user & (CB - 1))
    ipacked = ((item >> (SH + 1)) << SH) | (item & (CB - 1))
    uidx = upacked.reshape(NW, NCHUNK, CHUNK)
    iidx = ipacked.reshape(NW, NCHUNK, CHUNK)
    ue2, ie2 = _gather(uidx, iidx, ut_p, it_p)

    half = (jnp.arange(2 * D, dtype=jnp.int32) >= D)[None, :]
    mu = (half == ((user[:, None] & CB) != 0)).astype(jnp.float32)
    mi = (half == ((item[:, None] & CB) != 0)).astype(jnp.float32)

    s1 = g1 * lax.rsqrt(rv1 + EPS)
    a1 = (W1 * s1[:, None]).T
    c1 = ((b1 - rm1) * s1 + be1).reshape(1, H1)
    a1u = jnp.concatenate([a1[:D], a1[:D]], axis=0)
    a1i = jnp.concatenate([a1[D:], a1[D:]], axis=0)
    s2 = g2 * lax.rsqrt(rv2 + EPS)
    a2 = (W2 * s2[:, None]).T
    c2 = ((b2 - rm2) * s2 + be2).reshape(1, D)

    out = pl.pallas_call(
        _mlp_body,
        grid=(B // RB,),
        in_specs=[
            pl.BlockSpec((RB, 2 * D), lambda i: (i, 0)),
            pl.BlockSpec((RB, 2 * D), lambda i: (i, 0)),
            pl.BlockSpec((RB, 2 * D), lambda i: (i, 0)),
            pl.BlockSpec((RB, 2 * D), lambda i: (i, 0)),
            pl.BlockSpec((2 * D, H1), lambda i: (0, 0)),
            pl.BlockSpec((2 * D, H1), lambda i: (0, 0)),
            pl.BlockSpec((1, H1), lambda i: (0, 0)),
            pl.BlockSpec((H1, D), lambda i: (0, 0)),
            pl.BlockSpec((1, D), lambda i: (0, 0)),
            pl.BlockSpec((1, D), lambda i: (0, 0)),
            pl.BlockSpec((1, 1), lambda i: (0, 0)),
        ],
        out_specs=pl.BlockSpec((RB, 1), lambda i: (i, 0)),
        out_shape=jax.ShapeDtypeStruct((B, 1), jnp.float32),
    )(ue2, ie2, mu, mi, a1u, a1i, c1, a2, c2, W3, b3.reshape(1, 1))
    return out.reshape(B)


# speedup vs baseline: 3.4746x; 1.2083x over previous
import functools

import jax
import jax.numpy as jnp
from jax import lax
from jax.experimental import pallas as pl
from jax.experimental.pallas import tpu as pltpu
from jax.experimental.pallas import tpu_sc as plsc

B = 16384
D = 64
H1 = 128
EPS = 1e-5

NC = 2
NS = 16
NW = NC * NS
BPW = B // NW
CHUNK = 128
NCHUNK = BPW // CHUNK

CB = 8192
SH = CB.bit_length() - 1
RB = 2048

def _bf16_word(lo_bits, hi_bits):
    top = jnp.uint32(0xFFFF0000)
    hr = (hi_bits + jnp.uint32(0x7FFF) + ((hi_bits >> 16) & jnp.uint32(1)))
    lr = (lo_bits + jnp.uint32(0x7FFF) + ((lo_bits >> 16) & jnp.uint32(1)))
    return (hr & top) | (lr >> 16)


def _pack_body(b0, b1, b2, b3, out):
    quarters = []
    for blk in (b0, b1, b2, b3):
        x = lax.bitcast_convert_type(blk[...], jnp.uint32)
        quarters.append(_bf16_word(x[D // 2:], x[:D // 2]))
    w = jnp.concatenate(quarters, axis=0)
    out[...] = lax.bitcast_convert_type(w, jnp.float32).T


def _pack_table(table):
    n = table.shape[0]
    grid = (n + 4 * CB - 1) // (4 * CB)
    last = (n + CB - 1) // CB - 1
    tt = table.T
    return pl.pallas_call(
        _pack_body,
        grid=(grid,),
        in_specs=[
            pl.BlockSpec((D, CB), lambda g: (0, jnp.minimum(4 * g, last))),
            pl.BlockSpec((D, CB), lambda g: (0, jnp.minimum(4 * g + 1, last))),
            pl.BlockSpec((D, CB), lambda g: (0, jnp.minimum(4 * g + 2, last))),
            pl.BlockSpec((D, CB), lambda g: (0, jnp.minimum(4 * g + 3, last))),
        ],
        out_specs=pl.BlockSpec((CB, 2 * D), lambda g: (g, 0)),
        out_shape=jax.ShapeDtypeStruct((grid * CB, 2 * D), jnp.float32),
    )(tt, tt, tt, tt)


def _make_gather():
    mesh = plsc.VectorSubcoreMesh(core_axis_name="c", subcore_axis_name="s")

    @functools.partial(
        pl.kernel,
        mesh=mesh,
        compiler_params=pltpu.CompilerParams(use_tc_tiling_on_sc=False),
        out_type=[
            jax.ShapeDtypeStruct((B, 2 * D), jnp.float32),
            jax.ShapeDtypeStruct((B, 2 * D), jnp.float32),
        ],
        scratch_types=[
            pltpu.VMEM((NCHUNK, CHUNK), jnp.int32),
            pltpu.VMEM((NCHUNK, CHUNK), jnp.int32),
            pltpu.VMEM((BPW, 2 * D), jnp.float32),
            pltpu.SemaphoreType.DMA,
        ],
    )
    def gather(uidx_hbm, iidx_hbm, utab_hbm, itab_hbm, ue_hbm, ie_hbm,
               uidx_v, iidx_v, rows_v, sem):
        wid = lax.axis_index("s") * NC + lax.axis_index("c")
        base = wid * BPW
        pltpu.sync_copy(uidx_hbm.at[wid], uidx_v)
        pltpu.sync_copy(iidx_hbm.at[wid], iidx_v)
        ucopies = [
            pltpu.async_copy(utab_hbm.at[uidx_v.at[j]],
                             rows_v.at[pl.ds(j * CHUNK, CHUNK)], sem)
            for j in range(NCHUNK)
        ]
        for c in ucopies:
            c.wait()
        pltpu.sync_copy(rows_v, ue_hbm.at[pl.ds(base, BPW)])
        icopies = [
            pltpu.async_copy(itab_hbm.at[iidx_v.at[j]],
                             rows_v.at[pl.ds(j * CHUNK, CHUNK)], sem)
            for j in range(NCHUNK)
        ]
        for c in icopies:
            c.wait()
        pltpu.sync_copy(rows_v, ie_hbm.at[pl.ds(base, BPW)])

    return gather


_gather_cache = []


def _gather(*args):
    if not _gather_cache:
        _gather_cache.append(_make_gather())
    return _gather_cache[0](*args)


def _unpack(words_f32, mask_f32):
    w = lax.bitcast_convert_type(words_f32, jnp.uint32)
    keep = mask_f32 != 0.0
    top = jnp.uint32(0xFFFF0000)
    hi = jnp.where(keep, lax.bitcast_convert_type(w & top, jnp.float32),
                   0.0)
    lo = jnp.where(keep, lax.bitcast_convert_type(w << 16, jnp.float32), 0.0)
    return hi, lo


def _mlp_body(ue, ie, mu, mi, a1uh, a1ul, a1ih, a1il, c1, a2, c2, w3, b3,
              out):
    uh, ul = _unpack(ue[...], mu[...])
    ih, il = _unpack(ie[...], mi[...])
    h = jnp.dot(uh, a1uh[...], preferred_element_type=jnp.float32)
    h = h + jnp.dot(ul, a1ul[...], preferred_element_type=jnp.float32)
    h = h + jnp.dot(ih, a1ih[...], preferred_element_type=jnp.float32)
    h = h + jnp.dot(il, a1il[...], preferred_element_type=jnp.float32)
    h = jnp.maximum(h + c1[...], 0.0)
    h = jnp.dot(h, a2[...], preferred_element_type=jnp.float32)
    h = jnp.maximum(h + c2[...], 0.0)
    out[...] = jnp.sum(h * w3[...], axis=1, keepdims=True) + b3[...]


def kernel(user, item, user_table, item_table,
           W1, b1, g1, be1, rm1, rv1,
           W2, b2, g2, be2, rm2, rv2,
           W3, b3):
    user = user.astype(jnp.int32)
    item = item.astype(jnp.int32)

    ut_p = _pack_table(user_table)
    it_p = _pack_table(item_table)

    upacked = ((user >> (SH + 2)) << SH) | (user & (CB - 1))
    ipacked = ((item >> (SH + 2)) << SH) | (item & (CB - 1))
    uidx = upacked.reshape(NW, NCHUNK, CHUNK)
    iidx = ipacked.reshape(NW, NCHUNK, CHUNK)
    ue2, ie2 = _gather(uidx, iidx, ut_p, it_p)

    lanes = (jnp.arange(2 * D, dtype=jnp.int32) >> 5)[None, :]
    mu = (lanes == ((user >> SH) & 3)[:, None]).astype(jnp.float32)
    mi = (lanes == ((item >> SH) & 3)[:, None]).astype(jnp.float32)

    s1 = g1 * lax.rsqrt(rv1 + EPS)
    a1 = (W1 * s1[:, None]).T
    c1 = ((b1 - rm1) * s1 + be1).reshape(1, H1)
    hq = D // 2
    a1uh = jnp.tile(a1[0:hq], (4, 1))
    a1ul = jnp.tile(a1[hq:D], (4, 1))
    a1ih = jnp.tile(a1[D:D + hq], (4, 1))
    a1il = jnp.tile(a1[D + hq:], (4, 1))
    s2 = g2 * lax.rsqrt(rv2 + EPS)
    a2 = (W2 * s2[:, None]).T
    c2 = ((b2 - rm2) * s2 + be2).reshape(1, D)

    out = pl.pallas_call(
        _mlp_body,
        grid=(B // RB,),
        in_specs=[
            pl.BlockSpec((RB, 2 * D), lambda i: (i, 0)),
            pl.BlockSpec((RB, 2 * D), lambda i: (i, 0)),
            pl.BlockSpec((RB, 2 * D), lambda i: (i, 0)),
            pl.BlockSpec((RB, 2 * D), lambda i: (i, 0)),
            pl.BlockSpec((2 * D, H1), lambda i: (0, 0)),
            pl.BlockSpec((2 * D, H1), lambda i: (0, 0)),
            pl.BlockSpec((2 * D, H1), lambda i: (0, 0)),
            pl.BlockSpec((2 * D, H1), lambda i: (0, 0)),
            pl.BlockSpec((1, H1), lambda i: (0, 0)),
            pl.BlockSpec((H1, D), lambda i: (0, 0)),
            pl.BlockSpec((1, D), lambda i: (0, 0)),
            pl.BlockSpec((1, D), lambda i: (0, 0)),
            pl.BlockSpec((1, 1), lambda i: (0, 0)),
        ],
        out_specs=pl.BlockSpec((RB, 1), lambda i: (i, 0)),
        out_shape=jax.ShapeDtypeStruct((B, 1), jnp.float32),
    )(ue2, ie2, mu, mi, a1uh, a1ul, a1ih, a1il, c1, a2, c2, W3,
      b3.reshape(1, 1))
    return out.reshape(B)


# speedup vs baseline: 3.5814x; 1.0307x over previous
import functools

import jax
import jax.numpy as jnp
from jax import lax
from jax.experimental import pallas as pl
from jax.experimental.pallas import tpu as pltpu
from jax.experimental.pallas import tpu_sc as plsc

B = 16384
D = 64
H1 = 128
EPS = 1e-5

NC = 2
NS = 16
NW = NC * NS
BPW = B // NW
CHUNK = 128
NCHUNK = BPW // CHUNK

CB = 8192
SH = CB.bit_length() - 1
RB = 2048

def _bf16_word(lo_bits, hi_bits):
    top = jnp.uint32(0xFFFF0000)
    hr = (hi_bits + jnp.uint32(0x7FFF) + ((hi_bits >> 16) & jnp.uint32(1)))
    lr = (lo_bits + jnp.uint32(0x7FFF) + ((lo_bits >> 16) & jnp.uint32(1)))
    return (hr & top) | (lr >> 16)


def _pack_body(b0, b1, b2, b3, out):
    quarters = []
    for blk in (b0, b1, b2, b3):
        x = lax.bitcast_convert_type(blk[...], jnp.uint32)
        quarters.append(_bf16_word(x[D // 2:], x[:D // 2]))
    w = jnp.concatenate(quarters, axis=0)
    out[...] = lax.bitcast_convert_type(w, jnp.float32).T


def _pack_table(table):
    n = table.shape[0]
    grid = (n + 4 * CB - 1) // (4 * CB)
    last = (n + CB - 1) // CB - 1
    tt = table.T
    return pl.pallas_call(
        _pack_body,
        grid=(grid,),
        in_specs=[
            pl.BlockSpec((D, CB), lambda g: (0, jnp.minimum(4 * g, last))),
            pl.BlockSpec((D, CB), lambda g: (0, jnp.minimum(4 * g + 1, last))),
            pl.BlockSpec((D, CB), lambda g: (0, jnp.minimum(4 * g + 2, last))),
            pl.BlockSpec((D, CB), lambda g: (0, jnp.minimum(4 * g + 3, last))),
        ],
        out_specs=pl.BlockSpec((CB, 2 * D), lambda g: (g, 0)),
        out_shape=jax.ShapeDtypeStruct((grid * CB, 2 * D), jnp.float32),
    )(tt, tt, tt, tt)


def _make_gather():
    mesh = plsc.VectorSubcoreMesh(core_axis_name="c", subcore_axis_name="s")

    @functools.partial(
        pl.kernel,
        mesh=mesh,
        compiler_params=pltpu.CompilerParams(use_tc_tiling_on_sc=False),
        out_type=jax.ShapeDtypeStruct((B, 2 * D), jnp.float32),
        scratch_types=[
            pltpu.VMEM((NCHUNK, CHUNK), jnp.int32),
            pltpu.VMEM((BPW, 2 * D), jnp.float32),
            pltpu.SemaphoreType.DMA,
        ],
    )
    def gather(idx_hbm, tab_hbm, out_hbm, idx_v, rows_v, sem):
        wid = lax.axis_index("s") * NC + lax.axis_index("c")
        base = wid * BPW
        pltpu.sync_copy(idx_hbm.at[wid], idx_v)
        copies = [
            pltpu.async_copy(tab_hbm.at[idx_v.at[j]],
                             rows_v.at[pl.ds(j * CHUNK, CHUNK)], sem)
            for j in range(NCHUNK)
        ]
        for c in copies:
            c.wait()
        pltpu.sync_copy(rows_v, out_hbm.at[pl.ds(base, BPW)])

    return gather


_gather_cache = []


def _gather(*args):
    if not _gather_cache:
        _gather_cache.append(_make_gather())
    return _gather_cache[0](*args)


def _unpack(words_f32, keep):
    w = lax.bitcast_convert_type(words_f32, jnp.uint32)
    top = jnp.uint32(0xFFFF0000)
    hi = jnp.where(keep, lax.bitcast_convert_type(w & top, jnp.float32),
                   0.0)
    lo = jnp.where(keep, lax.bitcast_convert_type(w << 16, jnp.float32), 0.0)
    return hi, lo


def _mlp_body(ue, ie, uq, iq, a1uh, a1ul, a1ih, a1il, c1, a2, c2, w3, b3,
              out):
    laneq = lax.broadcasted_iota(jnp.int32, (1, 2 * D), 1) >> 5
    uh, ul = _unpack(ue[...], uq[...][0].T == laneq)
    ih, il = _unpack(ie[...], iq[...][0].T == laneq)
    h = jnp.dot(uh, a1uh[...], preferred_element_type=jnp.float32)
    h = h + jnp.dot(ul, a1ul[...], preferred_element_type=jnp.float32)
    h = h + jnp.dot(ih, a1ih[...], preferred_element_type=jnp.float32)
    h = h + jnp.dot(il, a1il[...], preferred_element_type=jnp.float32)
    h = jnp.maximum(h + c1[...], 0.0)
    h = jnp.dot(h, a2[...], preferred_element_type=jnp.float32)
    h = jnp.maximum(h + c2[...], 0.0)
    out[...] = jnp.sum(h * w3[...], axis=1, keepdims=True) + b3[...]


def kernel(user, item, user_table, item_table,
           W1, b1, g1, be1, rm1, rv1,
           W2, b2, g2, be2, rm2, rv2,
           W3, b3):
    user = user.astype(jnp.int32)
    item = item.astype(jnp.int32)

    it_p = _pack_table(item_table)
    ut_p = _pack_table(user_table)

    upacked = ((user >> (SH + 2)) << SH) | (user & (CB - 1))
    ipacked = ((item >> (SH + 2)) << SH) | (item & (CB - 1))
    uidx = upacked.reshape(NW, NCHUNK, CHUNK)
    iidx = ipacked.reshape(NW, NCHUNK, CHUNK)
    ie2 = _gather(iidx, it_p)
    ue2 = _gather(uidx, ut_p)

    uq = ((user >> SH) & 3).reshape(B // RB, 1, RB)
    iq = ((item >> SH) & 3).reshape(B // RB, 1, RB)

    s1 = g1 * lax.rsqrt(rv1 + EPS)
    a1 = (W1 * s1[:, None]).T
    c1 = ((b1 - rm1) * s1 + be1).reshape(1, H1)
    hq = D // 2
    a1uh = jnp.tile(a1[0:hq], (4, 1))
    a1ul = jnp.tile(a1[hq:D], (4, 1))
    a1ih = jnp.tile(a1[D:D + hq], (4, 1))
    a1il = jnp.tile(a1[D + hq:], (4, 1))
    s2 = g2 * lax.rsqrt(rv2 + EPS)
    a2 = (W2 * s2[:, None]).T
    c2 = ((b2 - rm2) * s2 + be2).reshape(1, D)

    out = pl.pallas_call(
        _mlp_body,
        grid=(B // RB,),
        in_specs=[
            pl.BlockSpec((RB, 2 * D), lambda i: (i, 0)),
            pl.BlockSpec((RB, 2 * D), lambda i: (i, 0)),
            pl.BlockSpec((1, 1, RB), lambda i: (i, 0, 0)),
            pl.BlockSpec((1, 1, RB), lambda i: (i, 0, 0)),
            pl.BlockSpec((2 * D, H1), lambda i: (0, 0)),
            pl.BlockSpec((2 * D, H1), lambda i: (0, 0)),
            pl.BlockSpec((2 * D, H1), lambda i: (0, 0)),
            pl.BlockSpec((2 * D, H1), lambda i: (0, 0)),
            pl.BlockSpec((1, H1), lambda i: (0, 0)),
            pl.BlockSpec((H1, D), lambda i: (0, 0)),
            pl.BlockSpec((1, D), lambda i: (0, 0)),
            pl.BlockSpec((1, D), lambda i: (0, 0)),
            pl.BlockSpec((1, 1), lambda i: (0, 0)),
        ],
        out_specs=pl.BlockSpec((RB, 1), lambda i: (i, 0)),
        out_shape=jax.ShapeDtypeStruct((B, 1), jnp.float32),
    )(ue2, ie2, uq, iq, a1uh, a1ul, a1ih, a1il, c1, a2, c2, W3,
      b3.reshape(1, 1))
    return out.reshape(B)


# speedup vs baseline: 3.5926x; 1.0031x over previous
import functools

import jax
import jax.numpy as jnp
from jax import lax
from jax.experimental import pallas as pl
from jax.experimental.pallas import tpu as pltpu
from jax.experimental.pallas import tpu_sc as plsc

B = 16384
D = 64
H1 = 128
EPS = 1e-5

NC = 2
NS = 16
NW = NC * NS
BPW = B // NW
CHUNK = 128
NCHUNK = BPW // CHUNK

CB = 16384
SH = CB.bit_length() - 1
RB = 4096

def _bf16_word(lo_bits, hi_bits):
    top = jnp.uint32(0xFFFF0000)
    hr = (hi_bits + jnp.uint32(0x7FFF) + ((hi_bits >> 16) & jnp.uint32(1)))
    lr = (lo_bits + jnp.uint32(0x7FFF) + ((lo_bits >> 16) & jnp.uint32(1)))
    return (hr & top) | (lr >> 16)


def _pack_body(b0, b1, b2, b3, out):
    quarters = []
    for blk in (b0, b1, b2, b3):
        x = lax.bitcast_convert_type(blk[...], jnp.uint32)
        quarters.append(_bf16_word(x[D // 2:], x[:D // 2]))
    w = jnp.concatenate(quarters, axis=0)
    out[...] = lax.bitcast_convert_type(w, jnp.float32).T


def _pack_table(table):
    n = table.shape[0]
    grid = (n + 4 * CB - 1) // (4 * CB)
    last = (n + CB - 1) // CB - 1
    tt = table.T
    return pl.pallas_call(
        _pack_body,
        grid=(grid,),
        in_specs=[
            pl.BlockSpec((D, CB), lambda g: (0, jnp.minimum(4 * g, last))),
            pl.BlockSpec((D, CB), lambda g: (0, jnp.minimum(4 * g + 1, last))),
            pl.BlockSpec((D, CB), lambda g: (0, jnp.minimum(4 * g + 2, last))),
            pl.BlockSpec((D, CB), lambda g: (0, jnp.minimum(4 * g + 3, last))),
        ],
        out_specs=pl.BlockSpec((CB, 2 * D), lambda g: (g, 0)),
        out_shape=jax.ShapeDtypeStruct((grid * CB, 2 * D), jnp.float32),
    )(tt, tt, tt, tt)


def _make_gather():
    mesh = plsc.VectorSubcoreMesh(core_axis_name="c", subcore_axis_name="s")

    @functools.partial(
        pl.kernel,
        mesh=mesh,
        compiler_params=pltpu.CompilerParams(use_tc_tiling_on_sc=False),
        out_type=jax.ShapeDtypeStruct((B, 2 * D), jnp.float32),
        scratch_types=[
            pltpu.VMEM((NCHUNK, CHUNK), jnp.int32),
            pltpu.VMEM((BPW, 2 * D), jnp.float32),
            pltpu.SemaphoreType.DMA,
        ],
    )
    def gather(idx_hbm, tab_hbm, out_hbm, idx_v, rows_v, sem):
        wid = lax.axis_index("s") * NC + lax.axis_index("c")
        base = wid * BPW
        pltpu.sync_copy(idx_hbm.at[wid], idx_v)
        copies = [
            pltpu.async_copy(tab_hbm.at[idx_v.at[j]],
                             rows_v.at[pl.ds(j * CHUNK, CHUNK)], sem)
            for j in range(NCHUNK)
        ]
        for c in copies:
            c.wait()
        pltpu.sync_copy(rows_v, out_hbm.at[pl.ds(base, BPW)])

    return gather


_gather_cache = []


def _gather(*args):
    if not _gather_cache:
        _gather_cache.append(_make_gather())
    return _gather_cache[0](*args)


def _unpack(words_f32, keep):
    w = lax.bitcast_convert_type(words_f32, jnp.uint32)
    top = jnp.uint32(0xFFFF0000)
    hi = jnp.where(keep, lax.bitcast_convert_type(w & top, jnp.float32),
                   0.0)
    lo = jnp.where(keep, lax.bitcast_convert_type(w << 16, jnp.float32), 0.0)
    return hi, lo


def _mlp_body(ue, ie, uq, iq, a1uh, a1ul, a1ih, a1il, c1, a2, c2, w3, b3,
              out):
    laneq = lax.broadcasted_iota(jnp.int32, (1, 2 * D), 1) >> 5
    uh, ul = _unpack(ue[...], uq[...][0].T == laneq)
    ih, il = _unpack(ie[...], iq[...][0].T == laneq)
    h = jnp.dot(uh, a1uh[...], preferred_element_type=jnp.float32)
    h = h + jnp.dot(ul, a1ul[...], preferred_element_type=jnp.float32)
    h = h + jnp.dot(ih, a1ih[...], preferred_element_type=jnp.float32)
    h = h + jnp.dot(il, a1il[...], preferred_element_type=jnp.float32)
    h = jnp.maximum(h + c1[...], 0.0)
    h = jnp.dot(h, a2[...], preferred_element_type=jnp.float32)
    h = jnp.maximum(h + c2[...], 0.0)
    out[...] = jnp.sum(h * w3[...], axis=1, keepdims=True) + b3[...]


def kernel(user, item, user_table, item_table,
           W1, b1, g1, be1, rm1, rv1,
           W2, b2, g2, be2, rm2, rv2,
           W3, b3):
    user = user.astype(jnp.int32)
    item = item.astype(jnp.int32)

    it_p = _pack_table(item_table)
    ut_p = _pack_table(user_table)

    upacked = ((user >> (SH + 2)) << SH) | (user & (CB - 1))
    ipacked = ((item >> (SH + 2)) << SH) | (item & (CB - 1))
    uidx = upacked.reshape(NW, NCHUNK, CHUNK)
    iidx = ipacked.reshape(NW, NCHUNK, CHUNK)
    ie2 = _gather(iidx, it_p)
    ue2 = _gather(uidx, ut_p)

    uq = ((user >> SH) & 3).reshape(B // RB, 1, RB)
    iq = ((item >> SH) & 3).reshape(B // RB, 1, RB)

    s1 = g1 * lax.rsqrt(rv1 + EPS)
    a1 = (W1 * s1[:, None]).T
    c1 = ((b1 - rm1) * s1 + be1).reshape(1, H1)
    hq = D // 2
    a1uh = jnp.tile(a1[0:hq], (4, 1))
    a1ul = jnp.tile(a1[hq:D], (4, 1))
    a1ih = jnp.tile(a1[D:D + hq], (4, 1))
    a1il = jnp.tile(a1[D + hq:], (4, 1))
    s2 = g2 * lax.rsqrt(rv2 + EPS)
    a2 = (W2 * s2[:, None]).T
    c2 = ((b2 - rm2) * s2 + be2).reshape(1, D)

    out = pl.pallas_call(
        _mlp_body,
        grid=(B // RB,),
        in_specs=[
            pl.BlockSpec((RB, 2 * D), lambda i: (i, 0)),
            pl.BlockSpec((RB, 2 * D), lambda i: (i, 0)),
            pl.BlockSpec((1, 1, RB), lambda i: (i, 0, 0)),
            pl.BlockSpec((1, 1, RB), lambda i: (i, 0, 0)),
            pl.BlockSpec((2 * D, H1), lambda i: (0, 0)),
            pl.BlockSpec((2 * D, H1), lambda i: (0, 0)),
            pl.BlockSpec((2 * D, H1), lambda i: (0, 0)),
            pl.BlockSpec((2 * D, H1), lambda i: (0, 0)),
            pl.BlockSpec((1, H1), lambda i: (0, 0)),
            pl.BlockSpec((H1, D), lambda i: (0, 0)),
            pl.BlockSpec((1, D), lambda i: (0, 0)),
            pl.BlockSpec((1, D), lambda i: (0, 0)),
            pl.BlockSpec((1, 1), lambda i: (0, 0)),
        ],
        out_specs=pl.BlockSpec((RB, 1), lambda i: (i, 0)),
        out_shape=jax.ShapeDtypeStruct((B, 1), jnp.float32),
    )(ue2, ie2, uq, iq, a1uh, a1ul, a1ih, a1il, c1, a2, c2, W3,
      b3.reshape(1, 1))
    return out.reshape(B)
